# Initial kernel scaffold; baseline (speedup 1.0000x reference)
import jax
import jax.numpy as jnp
from jax.experimental import pallas as pl


def kernel(x, pos, edge_index, kernel_points, W):
    raise NotImplementedError("write your pallas kernel here")


# speedup vs baseline: 50.8874x; 50.8874x over previous
import functools

import jax
import jax.numpy as jnp
from jax import lax
from jax.experimental import pallas as pl
from jax.experimental.pallas import tpu as pltpu
from jax.experimental.pallas import tpu_sc as plsc

SIGMA = 0.048
NEG_SLOPE = 0.1
L = 16
NTILES = 32


def _rsqrt(x):
    i = plsc.bitcast(x, jnp.int32)
    y = plsc.bitcast(jnp.int32(0x5F3759DF) - (i >> 1), jnp.float32)
    for _ in range(4):
        y = y * (1.5 - 0.5 * x * y * y)
    return y


def _sc_edge_kernel(N, E, C, K):
    EP = E // L
    BL = 2000
    NB = EP // BL
    GR = 128
    NSH = -(-N // (64 * 2 * L)) * (64 * 2 * L)
    NSH2 = NSH // 2
    NRT = NSH2 // L
    FC = NRT // 2
    NF = NRT // FC
    RCH = L

    mesh = plsc.VectorSubcoreMesh(core_axis_name="c", subcore_axis_name="s")

    def body(posx_h, posy_h, posz_h, src_h, dst_h, kp_h, z_h, out_h,
             posx, posy, posz, sbuf, dbuf, zb, obuf, wbuf, kpv, mxb,
             shared, sem):
        cid = lax.axis_index("c")
        sid = lax.axis_index("s")
        start = sid * EP
        base = cid * NSH2
        lanes = lax.iota(jnp.int32, L)

        pltpu.sync_copy(posx_h, posx)
        pltpu.sync_copy(posy_h, posy)
        pltpu.sync_copy(posz_h, posz)
        pltpu.sync_copy(kp_h, kpv)

        def zero_body(i, _):
            for c in range(C // L):
                zb[i, pl.ds(c * L, L)] = jnp.zeros((L,), jnp.float32)
            return 0
        lax.fori_loop(0, FC, zero_body, 0)
        rowbase = pl.multiple_of(sid * NRT, 8)
        for c in range(NF):
            pltpu.sync_copy(zb.at[pl.ds(0, FC)],
                            shared.at[pl.ds(pl.multiple_of(
                                rowbase + c * FC, 8), FC)])
        plsc.subcore_barrier()

        kpx = kpv[0, pl.ds(0, L)]
        kpy = kpv[1, pl.ds(0, L)]
        kpz = kpv[2, pl.ds(0, L)]
        kn2 = kpx * kpx + kpy * kpy + kpz * kpz
        kn2 = jnp.where((lanes >= 1) & (lanes <= K), kn2, 0.0)
        for s in (1, 2, 4, 8):
            mxb[pl.ds(0, L)] = kn2
            kn2 = jnp.maximum(kn2, plsc.load_gather(mxb, [lanes ^ s]))
        nrm = kn2 * _rsqrt(kn2 + 1e-30)
        gatev = nrm + SIGMA
        gate2v = gatev * gatev

        inv_sigma = jnp.float32(1.0 / SIGMA)

        def scan_body(i, _):
            sv = sbuf[pl.ds(i * L, L)]
            dv = dbuf[pl.ds(i * L, L)]
            ox = plsc.load_gather(posx, [sv]) - plsc.load_gather(posx, [dv])
            oy = plsc.load_gather(posy, [sv]) - plsc.load_gather(posy, [dv])
            oz = plsc.load_gather(posz, [sv]) - plsc.load_gather(posz, [dv])
            r2 = ox * ox + oy * oy + oz * oz
            m = (r2 <= gate2v) & (dv >= base) & (dv < base + NSH2)
            dloc = jnp.clip(dv - base, 0, NSH2 - 1)

            def heavy():
                cps = []
                for k in range(K):
                    kf = jnp.full((L,), k + 1, jnp.int32)
                    kx = plsc.load_gather(kpv,
                                          [jnp.zeros((L,), jnp.int32), kf])
                    ky = plsc.load_gather(kpv,
                                          [jnp.full((L,), 1, jnp.int32), kf])
                    kz = plsc.load_gather(kpv,
                                          [jnp.full((L,), 2, jnp.int32), kf])
                    dx = ox - kx
                    dy = oy - ky
                    dz = oz - kz
                    d2 = dx * dx + dy * dy + dz * dz + 1e-12
                    d = d2 * _rsqrt(d2)
                    w = jnp.maximum(0.0, 1.0 - d * inv_sigma)
                    w = jnp.where(m, w, 0.0)
                    wbuf[k, pl.ds(0, L)] = w
                    cps.append(pltpu.async_copy(z_h.at[sv * K + k],
                                                zb.at[pl.ds(k * L, L)], sem))
                for cp in cps:
                    cp.wait()

                def accum(r, _):
                    acc = [jnp.zeros((L,), jnp.float32)
                           for _ in range(C // L)]
                    for k in range(K):
                        wv = plsc.load_gather(
                            wbuf, [jnp.full((L,), k, jnp.int32),
                                   jnp.full((L,), r, jnp.int32)])
                        for c in range(C // L):
                            acc[c] = acc[c] + wv * zb[k * L + r,
                                                      pl.ds(c * L, L)]
                    for c in range(C // L):
                        obuf[r, pl.ds(c * L, L)] = acc[c]
                    return 0

                lax.fori_loop(0, RCH, accum, 0)
                pltpu.sync_copy(obuf, shared.at[dloc], add=True)

            lax.cond(jnp.any(m), heavy, lambda: None)
            return 0

        def blk_body(b, _):
            off = pl.multiple_of(start + b * BL, 8)
            pltpu.sync_copy(src_h.at[pl.ds(off, BL)], sbuf)
            pltpu.sync_copy(dst_h.at[pl.ds(off, BL)], dbuf)
            lax.fori_loop(0, BL // L, scan_body, 0)
            return 0

        lax.fori_loop(0, NB, blk_body, 0)
        plsc.subcore_barrier()

        for c in range(NF):
            pltpu.sync_copy(shared.at[pl.ds(pl.multiple_of(
                                rowbase + c * FC, 8), FC)],
                            zb.at[pl.ds(0, FC)])
            pltpu.sync_copy(zb.at[pl.ds(0, FC)],
                            out_h.at[pl.ds(pl.multiple_of(
                                base + rowbase + c * FC, 8), FC)])

    return pl.kernel(
        body,
        out_type=jax.ShapeDtypeStruct((NSH, C), jnp.float32),
        mesh=mesh,
        compiler_params=pltpu.CompilerParams(needs_layout_passes=False),
        scratch_types=[
            pltpu.VMEM((N,), jnp.float32),
            pltpu.VMEM((N,), jnp.float32),
            pltpu.VMEM((N,), jnp.float32),
            pltpu.VMEM((BL,), jnp.int32),
            pltpu.VMEM((BL,), jnp.int32),
            pltpu.VMEM((K * L, C), jnp.float32),
            pltpu.VMEM((RCH, C), jnp.float32),
            pltpu.VMEM((L, 128), jnp.float32),
            pltpu.VMEM((3, 128), jnp.float32),
            pltpu.VMEM((128,), jnp.float32),
            pltpu.VMEM_SHARED((NSH2, C), jnp.float32),
            pltpu.SemaphoreType.DMA,
        ],
    )


def _tc_matmul(x, wcat, N, C, K, bm=400):
    def body(x_ref, w_ref, o_ref):
        o_ref[...] = jnp.dot(x_ref[...], w_ref[...],
                             preferred_element_type=jnp.float32)

    return pl.pallas_call(
        body,
        grid=(N // bm,),
        in_specs=[
            pl.BlockSpec((bm, C), lambda i: (i, 0)),
            pl.BlockSpec((C, K * C), lambda i: (0, 0)),
        ],
        out_specs=pl.BlockSpec((bm, K * C), lambda i: (i, 0)),
        out_shape=jax.ShapeDtypeStruct((N, K * C), jnp.float32),
    )(x, wcat)


def _tc_combine(p0, N, C, bm=400):
    def body(a_ref, o_ref):
        s = a_ref[...]
        o_ref[...] = jnp.where(s >= 0, s, NEG_SLOPE * s)

    return pl.pallas_call(
        body,
        grid=(N // bm,),
        in_specs=[pl.BlockSpec((bm, C), lambda i: (i, 0))],
        out_specs=pl.BlockSpec((bm, C), lambda i: (i, 0)),
        out_shape=jax.ShapeDtypeStruct((N, C), jnp.float32),
    )(p0)


@jax.jit
def kernel(x, pos, edge_index, kernel_points, W):
    N, C = x.shape
    E = edge_index.shape[1]
    K = kernel_points.shape[0]

    wcat = jnp.transpose(W, (1, 0, 2)).reshape(C, K * C)
    kp_pad = jnp.full((3, 128), 1e3,
                      jnp.float32).at[:, 1:K + 1].set(kernel_points.T)
    src = edge_index[0]
    dst = edge_index[1]
    posx = pos[:, 0]
    posy = pos[:, 1]
    posz = pos[:, 2]

    z2 = _tc_matmul(x, wcat, N, C, K)
    zflat = z2.reshape(N * K, C)

    agg = _sc_edge_kernel(N, E, C, K)(
        posx, posy, posz, src, dst, kp_pad, zflat)

    return _tc_combine(agg[:N], N, C)


# speedup vs baseline: 55.4740x; 1.0901x over previous
import functools

import jax
import jax.numpy as jnp
from jax import lax
from jax.experimental import pallas as pl
from jax.experimental.pallas import tpu as pltpu
from jax.experimental.pallas import tpu_sc as plsc

SIGMA = 0.048
NEG_SLOPE = 0.1
L = 16
NTILES = 32


def _rsqrt(x):
    i = plsc.bitcast(x, jnp.int32)
    y = plsc.bitcast(jnp.int32(0x5F3759DF) - (i >> 1), jnp.float32)
    for _ in range(4):
        y = y * (1.5 - 0.5 * x * y * y)
    return y


def _sc_edge_kernel(N, E, C, K):
    EP = E // L
    BL = 2000
    NB = EP // BL
    GR = 128
    NSH = -(-N // (64 * 2 * L)) * (64 * 2 * L)
    NSH2 = NSH // 2
    NRT = NSH2 // L
    FC = NRT // 2
    NF = NRT // FC
    RCH = L

    mesh = plsc.VectorSubcoreMesh(core_axis_name="c", subcore_axis_name="s")

    def body(posx_h, posy_h, posz_h, src_h, dst_h, kp_h, z_h, out_h,
             posx, posy, posz, sbuf0, dbuf0, sbuf1, dbuf1, zb, obuf, wbuf,
             kpv, mxb, shared, sem, semA, semB):
        cid = lax.axis_index("c")
        sid = lax.axis_index("s")
        start = sid * EP
        base = cid * NSH2
        lanes = lax.iota(jnp.int32, L)

        st = [pltpu.async_copy(posx_h, posx, sem),
              pltpu.async_copy(posy_h, posy, sem),
              pltpu.async_copy(posz_h, posz, sem),
              pltpu.async_copy(kp_h, kpv, sem)]
        for cp in st:
            cp.wait()

        def zero_body(i, _):
            for c in range(C // L):
                zb[i, pl.ds(c * L, L)] = jnp.zeros((L,), jnp.float32)
            return 0
        lax.fori_loop(0, FC, zero_body, 0)
        rowbase = pl.multiple_of(sid * NRT, 8)
        for c in range(NF):
            pltpu.sync_copy(zb.at[pl.ds(0, FC)],
                            shared.at[pl.ds(pl.multiple_of(
                                rowbase + c * FC, 8), FC)])
        plsc.subcore_barrier()

        kpx = kpv[0, pl.ds(0, L)]
        kpy = kpv[1, pl.ds(0, L)]
        kpz = kpv[2, pl.ds(0, L)]
        kn2 = kpx * kpx + kpy * kpy + kpz * kpz
        kn2 = jnp.where((lanes >= 1) & (lanes <= K), kn2, 0.0)
        for s in (1, 2, 4, 8):
            mxb[pl.ds(0, L)] = kn2
            kn2 = jnp.maximum(kn2, plsc.load_gather(mxb, [lanes ^ s]))
        nrm = kn2 * _rsqrt(kn2 + 1e-30)
        gatev = nrm + SIGMA
        gate2v = gatev * gatev

        inv_sigma = jnp.float32(1.0 / SIGMA)

        def make_scan(sb, db):
            return functools.partial(scan_body, sb, db)

        def scan_body(sb, db, i, _):
            sv = sb[pl.ds(i * L, L)]
            dv = db[pl.ds(i * L, L)]
            ox = plsc.load_gather(posx, [sv]) - plsc.load_gather(posx, [dv])
            oy = plsc.load_gather(posy, [sv]) - plsc.load_gather(posy, [dv])
            oz = plsc.load_gather(posz, [sv]) - plsc.load_gather(posz, [dv])
            r2 = ox * ox + oy * oy + oz * oz
            m = (r2 <= gate2v) & (dv >= base) & (dv < base + NSH2)
            dloc = jnp.clip(dv - base, 0, NSH2 - 1)

            def heavy():
                cps = []
                for k in range(K):
                    kf = jnp.full((L,), k + 1, jnp.int32)
                    kx = plsc.load_gather(kpv,
                                          [jnp.zeros((L,), jnp.int32), kf])
                    ky = plsc.load_gather(kpv,
                                          [jnp.full((L,), 1, jnp.int32), kf])
                    kz = plsc.load_gather(kpv,
                                          [jnp.full((L,), 2, jnp.int32), kf])
                    dx = ox - kx
                    dy = oy - ky
                    dz = oz - kz
                    d2 = dx * dx + dy * dy + dz * dz + 1e-12
                    d = d2 * _rsqrt(d2)
                    w = jnp.maximum(0.0, 1.0 - d * inv_sigma)
                    w = jnp.where(m, w, 0.0)
                    wbuf[k, pl.ds(0, L)] = w
                    cps.append(pltpu.async_copy(z_h.at[sv * K + k],
                                                zb.at[pl.ds(k * L, L)], sem))
                for cp in cps:
                    cp.wait()

                def accum(r, _):
                    acc = [jnp.zeros((L,), jnp.float32)
                           for _ in range(C // L)]
                    for k in range(K):
                        wv = plsc.load_gather(
                            wbuf, [jnp.full((L,), k, jnp.int32),
                                   jnp.full((L,), r, jnp.int32)])
                        for c in range(C // L):
                            acc[c] = acc[c] + wv * zb[k * L + r,
                                                      pl.ds(c * L, L)]
                    for c in range(C // L):
                        obuf[r, pl.ds(c * L, L)] = acc[c]
                    return 0

                lax.fori_loop(0, RCH, accum, 0)
                pltpu.sync_copy(obuf, shared.at[dloc], add=True)

            lax.cond(jnp.any(m), heavy, lambda: None)
            return 0

        def fire(bidx, sb, db, sm):
            off = pl.multiple_of(
                start + jnp.minimum(bidx, NB - 1) * BL, 8)
            pltpu.async_copy(src_h.at[pl.ds(off, BL)], sb, sm)
            pltpu.async_copy(dst_h.at[pl.ds(off, BL)], db, sm)

        def drain(sb, db, sm):
            pltpu.make_async_copy(src_h.at[pl.ds(0, BL)], sb, sm).wait()
            pltpu.make_async_copy(dst_h.at[pl.ds(0, BL)], db, sm).wait()

        fire(jnp.int32(0), sbuf0, dbuf0, semA)

        def blk2(j, _):
            fire(j * 2 + 1, sbuf1, dbuf1, semB)
            drain(sbuf0, dbuf0, semA)
            lax.fori_loop(0, BL // L, make_scan(sbuf0, dbuf0), 0)
            fire(j * 2 + 2, sbuf0, dbuf0, semA)
            drain(sbuf1, dbuf1, semB)
            lax.fori_loop(0, BL // L, make_scan(sbuf1, dbuf1), 0)
            return 0

        lax.fori_loop(0, NB // 2, blk2, 0)
        drain(sbuf0, dbuf0, semA)
        plsc.subcore_barrier()

        for c in range(NF):
            pltpu.sync_copy(shared.at[pl.ds(pl.multiple_of(
                                rowbase + c * FC, 8), FC)],
                            zb.at[pl.ds(0, FC)])

            def lk(i, _):
                for cc in range(C // L):
                    v = zb[i, pl.ds(cc * L, L)]
                    zb[i, pl.ds(cc * L, L)] = jnp.maximum(v, NEG_SLOPE * v)
                return 0

            lax.fori_loop(0, FC, lk, 0)
            pltpu.sync_copy(zb.at[pl.ds(0, FC)],
                            out_h.at[pl.ds(pl.multiple_of(
                                base + rowbase + c * FC, 8), FC)])

    return pl.kernel(
        body,
        out_type=jax.ShapeDtypeStruct((NSH, C), jnp.float32),
        mesh=mesh,
        compiler_params=pltpu.CompilerParams(needs_layout_passes=False),
        scratch_types=[
            pltpu.VMEM((N,), jnp.float32),
            pltpu.VMEM((N,), jnp.float32),
            pltpu.VMEM((N,), jnp.float32),
            pltpu.VMEM((BL,), jnp.int32),
            pltpu.VMEM((BL,), jnp.int32),
            pltpu.VMEM((BL,), jnp.int32),
            pltpu.VMEM((BL,), jnp.int32),
            pltpu.VMEM((K * L, C), jnp.float32),
            pltpu.VMEM((RCH, C), jnp.float32),
            pltpu.VMEM((L, 128), jnp.float32),
            pltpu.VMEM((3, 128), jnp.float32),
            pltpu.VMEM((128,), jnp.float32),
            pltpu.VMEM_SHARED((NSH2, C), jnp.float32),
            pltpu.SemaphoreType.DMA,
            pltpu.SemaphoreType.DMA,
            pltpu.SemaphoreType.DMA,
        ],
    )


def _tc_matmul(x, wcat, N, C, K, bm=400):
    def body(x_ref, w_ref, o_ref):
        o_ref[...] = jnp.dot(x_ref[...], w_ref[...],
                             preferred_element_type=jnp.float32)

    return pl.pallas_call(
        body,
        grid=(N // bm,),
        in_specs=[
            pl.BlockSpec((bm, C), lambda i: (i, 0)),
            pl.BlockSpec((C, K * C), lambda i: (0, 0)),
        ],
        out_specs=pl.BlockSpec((bm, K * C), lambda i: (i, 0)),
        out_shape=jax.ShapeDtypeStruct((N, K * C), jnp.float32),
    )(x, wcat)


@jax.jit
def kernel(x, pos, edge_index, kernel_points, W):
    N, C = x.shape
    E = edge_index.shape[1]
    K = kernel_points.shape[0]

    wcat = jnp.transpose(W, (1, 0, 2)).reshape(C, K * C)
    kp_pad = jnp.full((3, 128), 1e3,
                      jnp.float32).at[:, 1:K + 1].set(kernel_points.T)
    src = edge_index[0]
    dst = edge_index[1]
    posx = pos[:, 0]
    posy = pos[:, 1]
    posz = pos[:, 2]

    z2 = _tc_matmul(x, wcat, N, C, K)
    zflat = z2.reshape(N * K, C)

    agg = _sc_edge_kernel(N, E, C, K)(
        posx, posy, posz, src, dst, kp_pad, zflat)

    return agg[:N]


# speedup vs baseline: 61.8687x; 1.1153x over previous
import functools

import jax
import jax.numpy as jnp
from jax import lax
from jax.experimental import pallas as pl
from jax.experimental.pallas import tpu as pltpu
from jax.experimental.pallas import tpu_sc as plsc

SIGMA = 0.048
NEG_SLOPE = 0.1
L = 16
NTILES = 32


def _rsqrt(x):
    i = plsc.bitcast(x, jnp.int32)
    y = plsc.bitcast(jnp.int32(0x5F3759DF) - (i >> 1), jnp.float32)
    for _ in range(4):
        y = y * (1.5 - 0.5 * x * y * y)
    return y


def _sc_edge_kernel(N, E, C, K):
    EP = E // L
    BL = 2000
    NB = EP // BL
    GR = 128
    NSH = -(-N // (64 * 2 * L)) * (64 * 2 * L)
    NSH2 = NSH // 2
    NRT = NSH2 // L
    FC = NRT // 2
    NF = NRT // FC
    RCH = L

    mesh = plsc.VectorSubcoreMesh(core_axis_name="c", subcore_axis_name="s")

    def body(posx_h, posy_h, posz_h, src_h, dst_h, kp_h, z_h, out_h,
             posx, posy, posz, sbuf0, dbuf0, sbuf1, dbuf1, zb, obuf, wbuf,
             kpv, mxb, shared, sem, semA, semB):
        cid = lax.axis_index("c")
        sid = lax.axis_index("s")
        start = sid * EP
        base = cid * NSH2
        lanes = lax.iota(jnp.int32, L)

        st = [pltpu.async_copy(posx_h, posx, sem),
              pltpu.async_copy(posy_h, posy, sem),
              pltpu.async_copy(posz_h, posz, sem),
              pltpu.async_copy(kp_h, kpv, sem)]
        for cp in st:
            cp.wait()

        def zero_body(i, _):
            for c in range(C // L):
                zb[i, pl.ds(c * L, L)] = jnp.zeros((L,), jnp.float32)
            return 0
        lax.fori_loop(0, FC, zero_body, 0)
        rowbase = pl.multiple_of(sid * NRT, 8)
        for c in range(NF):
            pltpu.sync_copy(zb.at[pl.ds(0, FC)],
                            shared.at[pl.ds(pl.multiple_of(
                                rowbase + c * FC, 8), FC)])
        plsc.subcore_barrier()

        kpx = kpv[0, pl.ds(0, L)]
        kpy = kpv[1, pl.ds(0, L)]
        kpz = kpv[2, pl.ds(0, L)]
        kn2 = kpx * kpx + kpy * kpy + kpz * kpz
        kn2 = jnp.where((lanes >= 1) & (lanes <= K), kn2, 0.0)
        for s in (1, 2, 4, 8):
            mxb[pl.ds(0, L)] = kn2
            kn2 = jnp.maximum(kn2, plsc.load_gather(mxb, [lanes ^ s]))
        nrm = kn2 * _rsqrt(kn2 + 1e-30)
        gatev = nrm + SIGMA
        gate2v = gatev * gatev

        inv_sigma = jnp.float32(1.0 / SIGMA)

        def make_scan(sb, db):
            return functools.partial(scan_body, sb, db)

        def scan_body(sb, db, i, _):
            sv = sb[pl.ds(i * L, L)]
            dv = db[pl.ds(i * L, L)]
            ox = plsc.load_gather(posx, [sv]) - plsc.load_gather(posx, [dv])
            oy = plsc.load_gather(posy, [sv]) - plsc.load_gather(posy, [dv])
            oz = plsc.load_gather(posz, [sv]) - plsc.load_gather(posz, [dv])
            r2 = ox * ox + oy * oy + oz * oz
            m = (r2 <= gate2v) & (dv >= base) & (dv < base + NSH2)
            dloc = jnp.clip(dv - base, 0, NSH2 - 1)

            def heavy():
                cps = []
                for k in range(K):
                    kf = jnp.full((L,), k + 1, jnp.int32)
                    kx = plsc.load_gather(kpv,
                                          [jnp.zeros((L,), jnp.int32), kf])
                    ky = plsc.load_gather(kpv,
                                          [jnp.full((L,), 1, jnp.int32), kf])
                    kz = plsc.load_gather(kpv,
                                          [jnp.full((L,), 2, jnp.int32), kf])
                    dx = ox - kx
                    dy = oy - ky
                    dz = oz - kz
                    d2 = dx * dx + dy * dy + dz * dz + 1e-12
                    d = d2 * _rsqrt(d2)
                    w = jnp.maximum(0.0, 1.0 - d * inv_sigma)
                    w = jnp.where(m, w, 0.0)
                    wbuf[k, pl.ds(0, L)] = w
                    cps.append(pltpu.async_copy(z_h.at[k * N + sv],
                                                zb.at[pl.ds(k * L, L)], sem))
                for cp in cps:
                    cp.wait()

                def accum(r, _):
                    acc = [jnp.zeros((L,), jnp.float32)
                           for _ in range(C // L)]
                    for k in range(K):
                        wv = plsc.load_gather(
                            wbuf, [jnp.full((L,), k, jnp.int32),
                                   jnp.full((L,), r, jnp.int32)])
                        for c in range(C // L):
                            acc[c] = acc[c] + wv * zb[k * L + r,
                                                      pl.ds(c * L, L)]
                    for c in range(C // L):
                        obuf[r, pl.ds(c * L, L)] = acc[c]
                    return 0

                lax.fori_loop(0, RCH, accum, 0)
                pltpu.sync_copy(obuf, shared.at[dloc], add=True)

            lax.cond(jnp.any(m), heavy, lambda: None)
            return 0

        def fire(bidx, sb, db, sm):
            off = pl.multiple_of(
                start + jnp.minimum(bidx, NB - 1) * BL, 8)
            pltpu.async_copy(src_h.at[pl.ds(off, BL)], sb, sm)
            pltpu.async_copy(dst_h.at[pl.ds(off, BL)], db, sm)

        def drain(sb, db, sm):
            pltpu.make_async_copy(src_h.at[pl.ds(0, BL)], sb, sm).wait()
            pltpu.make_async_copy(dst_h.at[pl.ds(0, BL)], db, sm).wait()

        fire(jnp.int32(0), sbuf0, dbuf0, semA)

        def blk2(j, _):
            fire(j * 2 + 1, sbuf1, dbuf1, semB)
            drain(sbuf0, dbuf0, semA)
            lax.fori_loop(0, BL // L, make_scan(sbuf0, dbuf0), 0)
            fire(j * 2 + 2, sbuf0, dbuf0, semA)
            drain(sbuf1, dbuf1, semB)
            lax.fori_loop(0, BL // L, make_scan(sbuf1, dbuf1), 0)
            return 0

        lax.fori_loop(0, NB // 2, blk2, 0)
        drain(sbuf0, dbuf0, semA)
        plsc.subcore_barrier()

        for c in range(NF):
            pltpu.sync_copy(shared.at[pl.ds(pl.multiple_of(
                                rowbase + c * FC, 8), FC)],
                            zb.at[pl.ds(0, FC)])

            def lk(i, _):
                for cc in range(C // L):
                    v = zb[i, pl.ds(cc * L, L)]
                    zb[i, pl.ds(cc * L, L)] = jnp.maximum(v, NEG_SLOPE * v)
                return 0

            lax.fori_loop(0, FC, lk, 0)
            pltpu.sync_copy(zb.at[pl.ds(0, FC)],
                            out_h.at[pl.ds(pl.multiple_of(
                                base + rowbase + c * FC, 8), FC)])

    return pl.kernel(
        body,
        out_type=jax.ShapeDtypeStruct((NSH, C), jnp.float32),
        mesh=mesh,
        compiler_params=pltpu.CompilerParams(needs_layout_passes=False),
        scratch_types=[
            pltpu.VMEM((N,), jnp.float32),
            pltpu.VMEM((N,), jnp.float32),
            pltpu.VMEM((N,), jnp.float32),
            pltpu.VMEM((BL,), jnp.int32),
            pltpu.VMEM((BL,), jnp.int32),
            pltpu.VMEM((BL,), jnp.int32),
            pltpu.VMEM((BL,), jnp.int32),
            pltpu.VMEM((K * L, C), jnp.float32),
            pltpu.VMEM((RCH, C), jnp.float32),
            pltpu.VMEM((L, 128), jnp.float32),
            pltpu.VMEM((3, 128), jnp.float32),
            pltpu.VMEM((128,), jnp.float32),
            pltpu.VMEM_SHARED((NSH2, C), jnp.float32),
            pltpu.SemaphoreType.DMA,
            pltpu.SemaphoreType.DMA,
            pltpu.SemaphoreType.DMA,
        ],
    )


def _tc_matmul(x, W, N, C, K, bm=2000):
    def body(x_ref, w_ref, o_ref):
        o_ref[0] = jnp.dot(x_ref[...], w_ref[0],
                           preferred_element_type=jnp.float32)

    return pl.pallas_call(
        body,
        grid=(K, N // bm),
        in_specs=[
            pl.BlockSpec((bm, C), lambda k, i: (i, 0)),
            pl.BlockSpec((1, C, C), lambda k, i: (k, 0, 0)),
        ],
        out_specs=pl.BlockSpec((1, bm, C), lambda k, i: (k, i, 0)),
        out_shape=jax.ShapeDtypeStruct((K, N, C), jnp.float32),
    )(x, W)


@jax.jit
def kernel(x, pos, edge_index, kernel_points, W):
    N, C = x.shape
    E = edge_index.shape[1]
    K = kernel_points.shape[0]

    kp_pad = jnp.full((3, 128), 1e3,
                      jnp.float32).at[:, 1:K + 1].set(kernel_points.T)
    src = edge_index[0]
    dst = edge_index[1]
    posT = pos.T
    posx = posT[0]
    posy = posT[1]
    posz = posT[2]

    z3 = _tc_matmul(x, W, N, C, K)
    zflat = z3.reshape(K * N, C)

    agg = _sc_edge_kernel(N, E, C, K)(
        posx, posy, posz, src, dst, kp_pad, zflat)

    return agg[:N]


# speedup vs baseline: 67.1309x; 1.0851x over previous
import functools

import jax
import jax.numpy as jnp
from jax import lax
from jax.experimental import pallas as pl
from jax.experimental.pallas import tpu as pltpu
from jax.experimental.pallas import tpu_sc as plsc

SIGMA = 0.048
NEG_SLOPE = 0.1
L = 16
NTILES = 32


def _rsqrt(x):
    i = plsc.bitcast(x, jnp.int32)
    y = plsc.bitcast(jnp.int32(0x5F3759DF) - (i >> 1), jnp.float32)
    for _ in range(4):
        y = y * (1.5 - 0.5 * x * y * y)
    return y


def _sc_edge_kernel(N, E, C, K):
    EP = E // L
    BL = 2000
    NB = EP // BL
    GR = 128
    NSH = -(-N // (64 * 2 * L)) * (64 * 2 * L)
    NSH2 = NSH // 2
    NRT = NSH2 // L
    FC = NRT // 2
    NF = NRT // FC
    RCH = L

    mesh = plsc.VectorSubcoreMesh(core_axis_name="c", subcore_axis_name="s")

    def body(posx_h, posy_h, posz_h, src_h, dst_h, kp_h, z_h, out_h,
             posx, posy, posz, sbuf0, dbuf0, sbuf1, dbuf1, zb, obuf, wbuf,
             kpv, mxb, shared, sem, semA, semB):
        cid = lax.axis_index("c")
        sid = lax.axis_index("s")
        start = sid * EP
        base = cid * NSH2
        lanes = lax.iota(jnp.int32, L)

        st = [pltpu.async_copy(posx_h, posx, sem),
              pltpu.async_copy(posy_h, posy, sem),
              pltpu.async_copy(posz_h, posz, sem),
              pltpu.async_copy(kp_h, kpv, sem)]
        for cp in st:
            cp.wait()

        def zero_body(i, _):
            for c in range(C // L):
                zb[i, pl.ds(c * L, L)] = jnp.zeros((L,), jnp.float32)
            return 0
        lax.fori_loop(0, FC, zero_body, 0)
        rowbase = pl.multiple_of(sid * NRT, 8)
        for c in range(NF):
            pltpu.sync_copy(zb.at[pl.ds(0, FC)],
                            shared.at[pl.ds(pl.multiple_of(
                                rowbase + c * FC, 8), FC)])
        plsc.subcore_barrier()

        kpx = kpv[0, pl.ds(0, L)]
        kpy = kpv[1, pl.ds(0, L)]
        kpz = kpv[2, pl.ds(0, L)]
        kn2 = kpx * kpx + kpy * kpy + kpz * kpz
        kn2 = jnp.where((lanes >= 1) & (lanes <= K), kn2, 0.0)
        for s in (1, 2, 4, 8):
            mxb[pl.ds(0, L)] = kn2
            kn2 = jnp.maximum(kn2, plsc.load_gather(mxb, [lanes ^ s]))
        nrm = kn2 * _rsqrt(kn2 + 1e-30)
        gatev = nrm + SIGMA
        gate2v = gatev * gatev

        inv_sigma = jnp.float32(1.0 / SIGMA)

        def make_scan(sb, db):
            return functools.partial(scan_body, sb, db)

        def scan_body(sb, db, i, _):
            sv = sb[pl.ds(i * L, L)]
            dv = db[pl.ds(i * L, L)]
            ox = plsc.load_gather(posx, [sv]) - plsc.load_gather(posx, [dv])
            oy = plsc.load_gather(posy, [sv]) - plsc.load_gather(posy, [dv])
            oz = plsc.load_gather(posz, [sv]) - plsc.load_gather(posz, [dv])
            r2 = ox * ox + oy * oy + oz * oz
            m = (r2 <= gate2v) & (dv >= base) & (dv < base + NSH2)
            dloc = jnp.clip(dv - base, 0, NSH2 - 1)

            def heavy():
                cps = []
                for k in range(K):
                    kf = jnp.full((L,), k + 1, jnp.int32)
                    kx = plsc.load_gather(kpv,
                                          [jnp.zeros((L,), jnp.int32), kf])
                    ky = plsc.load_gather(kpv,
                                          [jnp.full((L,), 1, jnp.int32), kf])
                    kz = plsc.load_gather(kpv,
                                          [jnp.full((L,), 2, jnp.int32), kf])
                    dx = ox - kx
                    dy = oy - ky
                    dz = oz - kz
                    d2 = dx * dx + dy * dy + dz * dz + 1e-12
                    d = d2 * _rsqrt(d2)
                    w = jnp.maximum(0.0, 1.0 - d * inv_sigma)
                    w = jnp.where(m, w, 0.0)
                    wbuf[k, pl.ds(0, L)] = w
                    cps.append(pltpu.async_copy(z_h.at[k * N + sv],
                                                zb.at[pl.ds(k * L, L)], sem))
                for cp in cps:
                    cp.wait()

                def accum(r, _):
                    acc = [jnp.zeros((L,), jnp.float32)
                           for _ in range(C // L)]
                    for k in range(K):
                        wv = plsc.load_gather(
                            wbuf, [jnp.full((L,), k, jnp.int32),
                                   jnp.full((L,), r, jnp.int32)])
                        for c in range(C // L):
                            acc[c] = acc[c] + wv * zb[k * L + r,
                                                      pl.ds(c * L, L)]
                    for c in range(C // L):
                        obuf[r, pl.ds(c * L, L)] = acc[c]
                    return 0

                lax.fori_loop(0, RCH, accum, 0)
                pltpu.sync_copy(obuf, shared.at[dloc], add=True)

            lax.cond(jnp.any(m), heavy, lambda: None)
            return 0

        def fire(bidx, sb, db, sm):
            off = pl.multiple_of(
                start + jnp.minimum(bidx, NB - 1) * BL, 8)
            pltpu.async_copy(src_h.at[pl.ds(off, BL)], sb, sm)
            pltpu.async_copy(dst_h.at[pl.ds(off, BL)], db, sm)

        def drain(sb, db, sm):
            pltpu.make_async_copy(src_h.at[pl.ds(0, BL)], sb, sm).wait()
            pltpu.make_async_copy(dst_h.at[pl.ds(0, BL)], db, sm).wait()

        fire(jnp.int32(0), sbuf0, dbuf0, semA)

        def blk2(j, _):
            fire(j * 2 + 1, sbuf1, dbuf1, semB)
            drain(sbuf0, dbuf0, semA)
            lax.fori_loop(0, BL // L, make_scan(sbuf0, dbuf0), 0)
            fire(j * 2 + 2, sbuf0, dbuf0, semA)
            drain(sbuf1, dbuf1, semB)
            lax.fori_loop(0, BL // L, make_scan(sbuf1, dbuf1), 0)
            return 0

        lax.fori_loop(0, NB // 2, blk2, 0)
        drain(sbuf0, dbuf0, semA)
        plsc.subcore_barrier()

        for c in range(NF):
            pltpu.sync_copy(shared.at[pl.ds(pl.multiple_of(
                                rowbase + c * FC, 8), FC)],
                            zb.at[pl.ds(0, FC)])

            def lk(i, _):
                for cc in range(C // L):
                    v = zb[i, pl.ds(cc * L, L)]
                    zb[i, pl.ds(cc * L, L)] = jnp.maximum(v, NEG_SLOPE * v)
                return 0

            lax.fori_loop(0, FC, lk, 0)
            pltpu.sync_copy(zb.at[pl.ds(0, FC)],
                            out_h.at[pl.ds(pl.multiple_of(
                                base + rowbase + c * FC, 8), FC)])

    return pl.kernel(
        body,
        out_type=jax.ShapeDtypeStruct((NSH, C), jnp.float32),
        mesh=mesh,
        compiler_params=pltpu.CompilerParams(needs_layout_passes=False),
        scratch_types=[
            pltpu.VMEM((N,), jnp.float32),
            pltpu.VMEM((N,), jnp.float32),
            pltpu.VMEM((N,), jnp.float32),
            pltpu.VMEM((BL,), jnp.int32),
            pltpu.VMEM((BL,), jnp.int32),
            pltpu.VMEM((BL,), jnp.int32),
            pltpu.VMEM((BL,), jnp.int32),
            pltpu.VMEM((K * L, C), jnp.float32),
            pltpu.VMEM((RCH, C), jnp.float32),
            pltpu.VMEM((L, 128), jnp.float32),
            pltpu.VMEM((3, 128), jnp.float32),
            pltpu.VMEM((128,), jnp.float32),
            pltpu.VMEM_SHARED((NSH2, C), jnp.float32),
            pltpu.SemaphoreType.DMA,
            pltpu.SemaphoreType.DMA,
            pltpu.SemaphoreType.DMA,
        ],
    )


def _tc_matmul(x, W, N, C, K, bm=2000):
    def body(x_ref, w_ref, o_ref):
        o_ref[0] = jnp.dot(x_ref[...], w_ref[0],
                           preferred_element_type=jnp.float32)

    return pl.pallas_call(
        body,
        grid=(N // bm, K),
        in_specs=[
            pl.BlockSpec((bm, C), lambda i, k: (i, 0)),
            pl.BlockSpec((1, C, C), lambda i, k: (k, 0, 0)),
        ],
        out_specs=pl.BlockSpec((1, bm, C), lambda i, k: (k, i, 0)),
        out_shape=jax.ShapeDtypeStruct((K, N, C), jnp.float32),
    )(x, W)


@jax.jit
def kernel(x, pos, edge_index, kernel_points, W):
    N, C = x.shape
    E = edge_index.shape[1]
    K = kernel_points.shape[0]

    kp_pad = jnp.full((3, 128), 1e3,
                      jnp.float32).at[:, 1:K + 1].set(kernel_points.T)
    src = edge_index[0]
    dst = edge_index[1]
    posT = pos.T
    posx = posT[0]
    posy = posT[1]
    posz = posT[2]

    z3 = _tc_matmul(x, W, N, C, K)
    zflat = z3.reshape(K * N, C)

    agg = _sc_edge_kernel(N, E, C, K)(
        posx, posy, posz, src, dst, kp_pad, zflat)

    return agg[:N]
